# baseline (speedup 1.0000x reference)
import jax
import jax.numpy as jnp
from jax import lax
from jax.experimental import pallas as pl
from jax.experimental.pallas import tpu as pltpu
from jax.experimental.pallas import tpu_sc as plsc

_LAMB = 0.7
_N = 10000
_D = 128
_LANES = 16

_NC = 2
_NS = 16
_NW = _NC * _NS
_PER_W = _N // _NW
_CHUNKS = (56, 56, 56, 56, 56, 32)
_OFFS = tuple(sum(_CHUNKS[:i]) for i in range(len(_CHUNKS)))
_NCHUNK = len(_CHUNKS)
_TAIL = _N - _NW * _PER_W


def _mix_rows(a_v, b_v, lo, hi):
    @plsc.parallel_loop(lo, hi, unroll=4)
    def _(i):
        for j in range(_D // _LANES):
            sl = pl.ds(j * _LANES, _LANES)
            a_v[i, sl] = a_v[i, sl] * _LAMB + b_v[i, sl] * (1.0 - _LAMB)


def _mix_body(x_hbm, pair_hbm, out_hbm, idx_v, idx_t, a_v, b_v, at_v, bt_v,
              sems, osem):
    wid = lax.axis_index("s") * _NC + lax.axis_index("c")
    base = wid * _PER_W

    pltpu.sync_copy(pair_hbm.at[pl.ds(base, _PER_W)], idx_v)
    copies = []
    for k in range(_NCHUNK):
        sl = pl.ds(_OFFS[k], _CHUNKS[k])
        copies.append((
            pltpu.async_copy(x_hbm.at[idx_v.at[sl]], b_v.at[sl], sems[k]),
            pltpu.async_copy(x_hbm.at[pl.ds(base + _OFFS[k], _CHUNKS[k])],
                             a_v.at[sl], sems[k]),
        ))

    stores = []
    for k in range(_NCHUNK):
        for c in copies[k]:
            c.wait()
        _mix_rows(a_v, b_v, _OFFS[k], _OFFS[k] + _CHUNKS[k])
        stores.append(pltpu.async_copy(
            a_v.at[pl.ds(_OFFS[k], _CHUNKS[k])],
            out_hbm.at[pl.ds(base + _OFFS[k], _CHUNKS[k])], osem))

    @pl.when(wid == _NW - 1)
    def _():
        tbase = _NW * _PER_W
        pltpu.sync_copy(pair_hbm.at[pl.ds(tbase, _TAIL)], idx_t)
        pltpu.async_copy(x_hbm.at[idx_t], bt_v, sems[0]).wait()
        pltpu.sync_copy(x_hbm.at[pl.ds(tbase, _TAIL)], at_v)
        _mix_rows(at_v, bt_v, 0, _TAIL)
        pltpu.sync_copy(at_v, out_hbm.at[pl.ds(tbase, _TAIL)])

    for s in stores:
        s.wait()


@jax.jit
def _node_mixup_sc(x, pair_idx):
    mesh = plsc.VectorSubcoreMesh(core_axis_name="c", subcore_axis_name="s")
    call = pl.kernel(
        _mix_body,
        out_type=jax.ShapeDtypeStruct((_N, _D), jnp.float32),
        mesh=mesh,
        scratch_types=[
            pltpu.VMEM((_PER_W,), jnp.int32),
            pltpu.VMEM((_TAIL,), jnp.int32),
            pltpu.VMEM((_PER_W, _D), jnp.float32),
            pltpu.VMEM((_PER_W, _D), jnp.float32),
            pltpu.VMEM((_TAIL, _D), jnp.float32),
            pltpu.VMEM((_TAIL, _D), jnp.float32),
            [pltpu.SemaphoreType.DMA] * _NCHUNK,
            pltpu.SemaphoreType.DMA,
        ],
    )
    return call(x, pair_idx)


def kernel(x, y, edge_index, pair_idx):
    x_mix = _node_mixup_sc(x, pair_idx)
    new_y = y.astype(jnp.int32)
    return (x_mix, new_y, edge_index)
